# baseline (speedup 1.0000x reference)
import jax
import jax.numpy as jnp
from jax import lax
from jax.experimental import pallas as pl
from jax.experimental.pallas import tpu as pltpu
from jax.experimental.pallas import tpu_sc as plsc

BATCH = 16384
LATENT = 32
ROWS_WIDE = 250000
WIDE = 128
NC = 2
NS = 16
NW = NC * NS
BPW = BATCH // NW
CH = 128
NCH = BPW // CH
L = 16
GROUPS_PER_CH = CH // L


def _body(inv_hbm, stk_hbm, w_inv_hbm, w_stk_hbm, out_hbm,
          idx_i, idx_s, div_i, div_s, rows_i, rows_s, out_v, sem):
    wid = lax.axis_index("s") * NC + lax.axis_index("c")
    base = wid * BPW

    pltpu.sync_copy(inv_hbm.at[pl.ds(base, BPW)], idx_i)
    pltpu.sync_copy(stk_hbm.at[pl.ds(base, BPW)], idx_s)
    for t in range(BPW // L):
        sl = pl.ds(t * L, L)
        div_i[sl] = jnp.right_shift(idx_i[sl], 2)
        div_s[sl] = jnp.right_shift(idx_s[sl], 2)

    lanes = lax.iota(jnp.int32, L)

    def compute_chunk(j):
        def g_body(g, carry):
            sl = pl.ds(pl.multiple_of(j * CH + g * L, L), L)
            off_i = jnp.bitwise_and(idx_i[sl], 3) * LATENT
            off_s = jnp.bitwise_and(idx_s[sl], 3) * LATENT
            r = g * L + lanes
            acc = jnp.zeros((L,), jnp.float32)
            for d in range(LATENT):
                a = plsc.load_gather(rows_i, [r, off_i + d])
                b = plsc.load_gather(rows_s, [r, off_s + d])
                acc = acc + a * b
            out_v[sl] = acc
            return carry
        lax.fori_loop(0, GROUPS_PER_CH, g_body, 0)

    for j in range(NCH):
        csl = pl.ds(j * CH, CH)
        ci = pltpu.async_copy(w_inv_hbm.at[div_i.at[csl]], rows_i, sem)
        cs = pltpu.async_copy(w_stk_hbm.at[div_s.at[csl]], rows_s, sem)
        ci.wait()
        cs.wait()
        compute_chunk(j)

    pltpu.sync_copy(out_v, out_hbm.at[pl.ds(base, BPW)])


@jax.jit
def kernel(investor, stock, W_investor, W_stock):
    w_inv = W_investor.reshape(ROWS_WIDE, WIDE)
    w_stk = W_stock.reshape(ROWS_WIDE, WIDE)
    mesh = plsc.VectorSubcoreMesh(core_axis_name="c", subcore_axis_name="s")
    return pl.kernel(
        _body,
        out_type=jax.ShapeDtypeStruct((BATCH,), jnp.float32),
        mesh=mesh,
        compiler_params=pltpu.CompilerParams(needs_layout_passes=False),
        scratch_types=[
            pltpu.VMEM((BPW,), jnp.int32),
            pltpu.VMEM((BPW,), jnp.int32),
            pltpu.VMEM((BPW,), jnp.int32),
            pltpu.VMEM((BPW,), jnp.int32),
            pltpu.VMEM((CH, WIDE), jnp.float32),
            pltpu.VMEM((CH, WIDE), jnp.float32),
            pltpu.VMEM((BPW,), jnp.float32),
            pltpu.SemaphoreType.DMA,
        ],
    )(investor, stock, w_inv, w_stk)


# speedup vs baseline: 3.7225x; 3.7225x over previous
import jax
import jax.numpy as jnp
from jax import lax
from jax.experimental import pallas as pl
from jax.experimental.pallas import tpu as pltpu
from jax.experimental.pallas import tpu_sc as plsc

BATCH = 16384
LATENT = 32
ROWS = 1000000
NC = 2
NS = 16
NW = NC * NS
BPW = BATCH // NW
L = 16
EB = 8
NBLK = BPW // EB
NGROUPS = BPW // L
PAD = L


def _body(inv_hbm, stk_hbm, w_inv_hbm, w_stk_hbm, out_hbm,
          idx_i, idx_s, buf_i, buf_s, pk_i, pk_s, out_v, sem_i, sem_s):
    wid = lax.axis_index("s") * NC + lax.axis_index("c")
    base = wid * BPW

    pltpu.sync_copy(inv_hbm.at[pl.ds(base, BPW)], idx_i.at[pl.ds(0, BPW)])
    pltpu.sync_copy(stk_hbm.at[pl.ds(base, BPW)], idx_s.at[pl.ds(0, BPW)])

    lanes = lax.iota(jnp.int32, L)
    rep8 = jnp.bitwise_and(lanes, 7)
    half = jnp.right_shift(lanes, 3)
    row_pat = rep8 * LATENT + half

    def fire(w_hbm, idx_ref, buf, sem, b):
        v = idx_ref[pl.ds(pl.multiple_of(b * EB, EB), L)]
        for j in range(EB):
            cb = jnp.right_shift(v[j], 7)
            src = w_hbm.at[:, pl.ds(pl.multiple_of(cb * 128, 128), 128)]
            pltpu.async_copy(src, buf.at[pl.ds(j * LATENT, LATENT), :], sem)

    def drain(w_hbm, buf, sem):
        for j in range(EB):
            pltpu.make_async_copy(
                w_hbm.at[:, pl.ds(0, 128)],
                buf.at[pl.ds(0, LATENT), :], sem).wait()

    def extract(idx_ref, buf, pk, b):
        v = idx_ref[pl.ds(pl.multiple_of(b * EB, EB), L)]
        col16 = jnp.bitwise_and(v, 127)
        colrep = lax.gather(
            col16, rep8.reshape(L, 1),
            lax.GatherDimensionNumbers(
                offset_dims=(), collapsed_slice_dims=(0,),
                start_index_map=(0,)),
            (1,), mode=lax.GatherScatterMode.PROMISE_IN_BOUNDS)
        for step in range(L):
            dbase = step * 2
            rows = row_pat + dbase
            val = plsc.load_gather(buf, [rows, colrep])
            pk[pl.ds(pl.multiple_of(b * (EB * LATENT) + step * L, L), L)] = val

    fire(w_inv_hbm, idx_i, buf_i, sem_i, 0)
    fire(w_stk_hbm, idx_s, buf_s, sem_s, 0)

    def pipe(b, carry):
        drain(w_inv_hbm, buf_i, sem_i)
        extract(idx_i, buf_i, pk_i, b)

        @pl.when(b < NBLK - 1)
        def _():
            fire(w_inv_hbm, idx_i, buf_i, sem_i, b + 1)

        drain(w_stk_hbm, buf_s, sem_s)
        extract(idx_s, buf_s, pk_s, b)

        @pl.when(b < NBLK - 1)
        def _():
            fire(w_stk_hbm, idx_s, buf_s, sem_s, b + 1)

        return carry

    lax.fori_loop(0, NBLK, pipe, 0)

    gbase_pat = half * (EB * LATENT) + rep8

    def g_body(g, carry):
        kbase = gbase_pat + g * (2 * EB * LATENT)
        acc = jnp.zeros((L,), jnp.float32)
        for d in range(LATENT):
            koff = (d // 2) * L + (d % 2) * EB
            a = plsc.load_gather(pk_i, [kbase + koff])
            b2 = plsc.load_gather(pk_s, [kbase + koff])
            acc = acc + a * b2
        out_v[pl.ds(pl.multiple_of(g * L, L), L)] = acc
        return carry

    lax.fori_loop(0, NGROUPS, g_body, 0)

    pltpu.sync_copy(out_v, out_hbm.at[pl.ds(base, BPW)])


@jax.jit
def kernel(investor, stock, W_investor, W_stock):
    w_inv = W_investor.T
    w_stk = W_stock.T
    mesh = plsc.VectorSubcoreMesh(core_axis_name="c", subcore_axis_name="s")
    return pl.kernel(
        _body,
        out_type=jax.ShapeDtypeStruct((BATCH,), jnp.float32),
        mesh=mesh,
        compiler_params=pltpu.CompilerParams(needs_layout_passes=False),
        scratch_types=[
            pltpu.VMEM((BPW + PAD,), jnp.int32),
            pltpu.VMEM((BPW + PAD,), jnp.int32),
            pltpu.VMEM((EB * LATENT, 128), jnp.float32),
            pltpu.VMEM((EB * LATENT, 128), jnp.float32),
            pltpu.VMEM((BPW * LATENT,), jnp.float32),
            pltpu.VMEM((BPW * LATENT,), jnp.float32),
            pltpu.VMEM((BPW,), jnp.float32),
            pltpu.SemaphoreType.DMA,
            pltpu.SemaphoreType.DMA,
        ],
    )(investor, stock, w_inv, w_stk)


# speedup vs baseline: 3.7944x; 1.0193x over previous
import jax
import jax.numpy as jnp
from jax import lax
from jax.experimental import pallas as pl
from jax.experimental.pallas import tpu as pltpu
from jax.experimental.pallas import tpu_sc as plsc

BATCH = 16384
LATENT = 32
NC = 2
NS = 16
NW = NC * NS
BPW = BATCH // NW
L = 16
EB = 4
NBLK = BPW // EB
NBLKP = NBLK + 4
DEPTH = 3
NSUP = 43
OPAD = 32


def _gd(x, idx):
    return lax.gather(
        x, idx.reshape(L, 1),
        lax.GatherDimensionNumbers(
            offset_dims=(), collapsed_slice_dims=(0,), start_index_map=(0,)),
        (1,), mode=lax.GatherScatterMode.PROMISE_IN_BOUNDS)


def _body(inv_hbm, stk_hbm, w_inv_hbm, w_stk_hbm, out_hbm,
          idx_i, idx_s, idxp_i, idxp_s, buf_i, buf_s, out_v,
          s0i, s0s, s1i, s1s, s2i, s2s):
    wid = lax.axis_index("s") * NC + lax.axis_index("c")
    base = wid * BPW
    lanes = lax.iota(jnp.int32, L)
    e4 = jnp.bitwise_and(lanes, 3)
    q4 = jnp.right_shift(lanes, 2)
    row_pat = e4 * LATENT + q4

    pltpu.sync_copy(inv_hbm.at[pl.ds(base, BPW)], idx_i)
    pltpu.sync_copy(stk_hbm.at[pl.ds(base, BPW)], idx_s)

    zero = jnp.zeros((L,), jnp.int32)
    for t in range(4):
        idxp_i[pl.ds((NBLK + t) * L, L)] = zero
        idxp_s[pl.ds((NBLK + t) * L, L)] = zero

    def stage_body(v, carry):
        xi = idx_i[pl.ds(pl.multiple_of(v * L, L), L)]
        xs = idx_s[pl.ds(pl.multiple_of(v * L, L), L)]
        for k in range(4):
            perm = e4 + 4 * k
            sl = pl.ds(pl.multiple_of((v * 4 + k) * L, L), L)
            idxp_i[sl] = _gd(xi, perm)
            idxp_s[sl] = _gd(xs, perm)
        return carry

    lax.fori_loop(0, BPW // L, stage_body, 0)

    sems = [(s0i, s0s), (s1i, s1s), (s2i, s2s)]

    def fire(b, slot):
        vi = idxp_i[pl.ds(pl.multiple_of(b * L, L), L)]
        vs = idxp_s[pl.ds(pl.multiple_of(b * L, L), L)]
        smi, sms = sems[slot]
        for j in range(EB):
            cbi = jnp.right_shift(vi[j], 7)
            cbs = jnp.right_shift(vs[j], 7)
            dsl = pl.ds((slot * EB + j) * LATENT, LATENT)
            pltpu.async_copy(
                w_inv_hbm.at[:, pl.ds(pl.multiple_of(cbi * 128, 128), 128)],
                buf_i.at[dsl, :], smi)
            pltpu.async_copy(
                w_stk_hbm.at[:, pl.ds(pl.multiple_of(cbs * 128, 128), 128)],
                buf_s.at[dsl, :], sms)

    def drain(slot):
        smi, sms = sems[slot]
        for j in range(EB):
            pltpu.make_async_copy(
                w_inv_hbm.at[:, pl.ds(0, 128)],
                buf_i.at[pl.ds(0, LATENT), :], smi).wait()
            pltpu.make_async_copy(
                w_stk_hbm.at[:, pl.ds(0, 128)],
                buf_s.at[pl.ds(0, LATENT), :], sms).wait()

    def extract_fma(b, slot, carry):
        vi = idxp_i[pl.ds(pl.multiple_of(b * L, L), L)]
        vs = idxp_s[pl.ds(pl.multiple_of(b * L, L), L)]
        col_i = jnp.bitwise_and(_gd(vi, e4), 127)
        col_s = jnp.bitwise_and(_gd(vs, e4), 127)
        rows0 = row_pat + slot * (EB * LATENT)
        acc = jnp.zeros((L,), jnp.float32)
        for step in range(8):
            rows = rows0 + step * EB
            a = plsc.load_gather(buf_i, [rows, col_i])
            b2 = plsc.load_gather(buf_s, [rows, col_s])
            acc = acc + a * b2
        t1 = acc + _gd(acc, jnp.bitwise_and(lanes + 8, 15))
        t2 = t1 + _gd(t1, jnp.bitwise_and(lanes + 4, 15))
        ph = jnp.bitwise_and(b, 3)
        sh = _gd(t2, jnp.bitwise_and(lanes - ph * 4, 15))
        sel = q4 == ph
        carry = jnp.where(sel, sh, carry)
        obase = pl.multiple_of(jnp.right_shift(b, 2) * L, L)
        out_v[pl.ds(obase, L)] = carry
        return carry

    fire(0, 0)
    fire(1, 1)
    fire(2, 2)

    def super_body(m, carry):
        for k in range(DEPTH):
            b = m * DEPTH + k
            drain(k)
            carry = extract_fma(b, k, carry)
            fire(b + DEPTH, k)
        return carry

    lax.fori_loop(0, NSUP, super_body, jnp.zeros((L,), jnp.float32))
    for k in range(DEPTH):
        drain(k)

    pltpu.sync_copy(out_v.at[pl.ds(0, BPW)], out_hbm.at[pl.ds(base, BPW)])


@jax.jit
def kernel(investor, stock, W_investor, W_stock):
    w_inv = W_investor.T
    w_stk = W_stock.T
    mesh = plsc.VectorSubcoreMesh(core_axis_name="c", subcore_axis_name="s")
    return pl.kernel(
        _body,
        out_type=jax.ShapeDtypeStruct((BATCH,), jnp.float32),
        mesh=mesh,
        compiler_params=pltpu.CompilerParams(needs_layout_passes=False),
        scratch_types=[
            pltpu.VMEM((BPW,), jnp.int32),
            pltpu.VMEM((BPW,), jnp.int32),
            pltpu.VMEM((NBLKP * L,), jnp.int32),
            pltpu.VMEM((NBLKP * L,), jnp.int32),
            pltpu.VMEM((DEPTH * EB * LATENT, 128), jnp.float32),
            pltpu.VMEM((DEPTH * EB * LATENT, 128), jnp.float32),
            pltpu.VMEM((BPW + OPAD,), jnp.float32),
            pltpu.SemaphoreType.DMA,
            pltpu.SemaphoreType.DMA,
            pltpu.SemaphoreType.DMA,
            pltpu.SemaphoreType.DMA,
            pltpu.SemaphoreType.DMA,
            pltpu.SemaphoreType.DMA,
        ],
    )(investor, stock, w_inv, w_stk)
